# baseline (speedup 1.0000x reference)
import functools

import jax
import jax.numpy as jnp
from jax import lax
from jax.experimental import pallas as pl
from jax.experimental.pallas import tpu as pltpu
from jax.experimental.pallas import tpu_sc as plsc

_J, _K = 8, 7


def _sc_body(x_hbm, o_hbm, sem):
    info = plsc.get_sparse_core_info()
    nc, ns = info.num_cores, info.num_subcores
    nw = nc * ns
    n_tasks = 8 * _J * _K
    assert n_tasks % nw == 0
    per_w = n_tasks // nw
    wid = lax.axis_index("s") * nc + lax.axis_index("c")
    copies = []
    for i in range(per_w):
        t = wid * per_w + i
        b = t // (_J * _K)
        r = t - b * (_J * _K)
        j = r // _K
        k = r - j * _K
        row = jnp.where(k < 6, k, 6 + j)
        c = pltpu.make_async_copy(x_hbm.at[b, row], o_hbm.at[b, j, k], sem)
        c.start()
        copies.append(c)
    for c in copies:
        c.wait()


def kernel(x):
    b, s, n, m, d = x.shape
    mesh = plsc.VectorSubcoreMesh(core_axis_name="c", subcore_axis_name="s")
    fn = pl.kernel(
        _sc_body,
        out_type=jax.ShapeDtypeStruct((b, _J, _K, n, m, d), x.dtype),
        mesh=mesh,
        scratch_types=[pltpu.SemaphoreType.DMA],
    )
    return fn(x)


# speedup vs baseline: 13.7427x; 13.7427x over previous
import jax
import jax.numpy as jnp
from jax import lax
from jax.experimental import pallas as pl
from jax.experimental.pallas import tpu as pltpu
from jax.experimental.pallas import tpu_sc as plsc

_ROW = 128 * 6 * 128
_Q = 4
_C = _ROW // _Q
_NBUF = 4
_J, _K = 8, 7


def _dests(r):
    return [(j, r) for j in range(_J)] if r < 6 else [(r - 6, 6)]


def _sc_body(x_hbm, o_hbm, *scratch):
    bufs = scratch[:_NBUF]
    lsems = scratch[_NBUF:2 * _NBUF]
    ssems = scratch[2 * _NBUF:]
    info = plsc.get_sparse_core_info()
    nc = info.num_cores
    wid = lax.axis_index("s") * nc + lax.axis_index("c")
    b = wid // _Q
    base = (wid % _Q) * _C

    def load(r):
        src = x_hbm.at[pl.ds((b * 14 + r) * _ROW + base, _C)]
        return pltpu.make_async_copy(src, bufs[r % _NBUF], lsems[r % _NBUF])

    def store(r, j, k):
        dst = o_hbm.at[pl.ds(((b * _J + j) * _K + k) * _ROW + base, _C)]
        return pltpu.make_async_copy(bufs[r % _NBUF], dst, ssems[r % _NBUF])

    for r in range(_NBUF):
        load(r).start()
    for r in range(14):
        load(r).wait()
        stores = [store(r, j, k) for (j, k) in _dests(r)]
        for st in stores:
            st.start()
        if r + _NBUF < 14:
            for st in stores:
                st.wait()
            load(r + _NBUF).start()
    for r in range(14 - _NBUF, 14):
        for st in [store(r, j, k) for (j, k) in _dests(r)]:
            st.wait()


def kernel(x):
    b, s, n, m, d = x.shape
    mesh = plsc.VectorSubcoreMesh(core_axis_name="c", subcore_axis_name="s")
    fn = pl.kernel(
        _sc_body,
        out_type=jax.ShapeDtypeStruct((b * _J * _K * _ROW,), x.dtype),
        mesh=mesh,
        scratch_types=(
            [pltpu.VMEM((_C,), x.dtype)] * _NBUF
            + [pltpu.SemaphoreType.DMA] * (2 * _NBUF)
        ),
    )
    out = fn(x.reshape(-1))
    return out.reshape(b, _J, _K, n, m, d)


# speedup vs baseline: 13.8101x; 1.0049x over previous
import jax
import jax.numpy as jnp
from jax import lax
from jax.experimental import pallas as pl
from jax.experimental.pallas import tpu as pltpu
from jax.experimental.pallas import tpu_sc as plsc

_ROW = 128 * 6 * 128
_J, _K = 8, 7


def _sc_body(x_hbm, o_hbm, buf, lsem, ssem):
    info = plsc.get_sparse_core_info()
    nc = info.num_cores
    wid = lax.axis_index("s") * nc + lax.axis_index("c")

    def row_copy(src_row, dst_rows):
        ld = pltpu.make_async_copy(
            x_hbm.at[pl.ds(src_row * _ROW, _ROW)], buf, lsem)
        ld.start()
        ld.wait()
        stores = [
            pltpu.make_async_copy(
                buf, o_hbm.at[pl.ds(dst * _ROW, _ROW)], ssem)
            for dst in dst_rows
        ]
        for st in stores:
            st.start()
        for st in stores:
            st.wait()

    def bgroup(g):
        b, k = g // 6, g % 6
        row_copy(b * 14 + k, [(b * _J + j) * _K + k for j in range(_J)])

    def diag(t):
        b, j = t // _J, t % _J
        row_copy(b * 14 + 6 + j, [(b * _J + j) * _K + 6])

    @pl.when(wid < 16)
    def _():
        for i in range(2):
            bgroup(wid * 2 + i)

    @pl.when(wid >= 16)
    def _():
        bgroup(32 + (wid - 16))
        for i in range(4):
            diag((wid - 16) * 4 + i)


def kernel(x):
    b, s, n, m, d = x.shape
    mesh = plsc.VectorSubcoreMesh(core_axis_name="c", subcore_axis_name="s")
    fn = pl.kernel(
        _sc_body,
        out_type=jax.ShapeDtypeStruct((b * _J * _K * _ROW,), x.dtype),
        mesh=mesh,
        scratch_types=[
            pltpu.VMEM((_ROW,), x.dtype),
            pltpu.SemaphoreType.DMA,
            pltpu.SemaphoreType.DMA,
        ],
    )
    out = fn(x.reshape(-1))
    return out.reshape(b, _J, _K, n, m, d)


# speedup vs baseline: 27.0775x; 1.9607x over previous
import jax
import jax.numpy as jnp
from jax import lax
from jax.experimental import pallas as pl
from jax.experimental.pallas import tpu as pltpu
from jax.experimental.pallas import tpu_sc as plsc

_J, _K = 8, 7


def _sc_body(x_hbm, o_hbm, buf, lsem, ssem):
    info = plsc.get_sparse_core_info()
    nc = info.num_cores
    wid = lax.axis_index("s") * nc + lax.axis_index("c")

    def row_copy(src, dsts):
        b, r = src
        ld = pltpu.make_async_copy(x_hbm.at[b, r], buf, lsem)
        ld.start()
        ld.wait()
        stores = [
            pltpu.make_async_copy(buf, o_hbm.at[bb, j, k], ssem)
            for (bb, j, k) in dsts
        ]
        for st in stores:
            st.start()
        for st in stores:
            st.wait()

    def bgroup(g):
        b, k = g // 6, g % 6
        row_copy((b, k), [(b, j, k) for j in range(_J)])

    def diag(t):
        b, j = t // _J, t % _J
        row_copy((b, 6 + j), [(b, j, 6)])

    @pl.when(wid < 16)
    def _():
        for i in range(2):
            bgroup(wid * 2 + i)

    @pl.when(wid >= 16)
    def _():
        bgroup(32 + (wid - 16))
        for i in range(4):
            diag((wid - 16) * 4 + i)


def kernel(x):
    b, s, n, m, d = x.shape
    mesh = plsc.VectorSubcoreMesh(core_axis_name="c", subcore_axis_name="s")
    fn = pl.kernel(
        _sc_body,
        out_type=jax.ShapeDtypeStruct((b, _J, _K, n, m, d), x.dtype),
        mesh=mesh,
        scratch_types=[
            pltpu.VMEM((n, m, d), x.dtype),
            pltpu.SemaphoreType.DMA,
            pltpu.SemaphoreType.DMA,
        ],
    )
    return fn(x)


# speedup vs baseline: 27.1317x; 1.0020x over previous
import jax
import jax.numpy as jnp
from jax import lax
from jax.experimental import pallas as pl
from jax.experimental.pallas import tpu as pltpu
from jax.experimental.pallas import tpu_sc as plsc

_J, _K = 8, 7


def _sc_body(x_hbm, o_hbm, buf, lsem, ssem):
    info = plsc.get_sparse_core_info()
    nc = info.num_cores
    wid = lax.axis_index("s") * nc + lax.axis_index("c")

    def row_copy(src, dsts):
        b, r = src
        ld = pltpu.make_async_copy(x_hbm.at[b, r], buf, lsem)
        ld.start()
        ld.wait()
        stores = [
            pltpu.make_async_copy(buf, o_hbm.at[bb, j, k], ssem)
            for (bb, j, k) in dsts
        ]
        for st in stores:
            st.start()
        for st in stores:
            st.wait()

    def bgroup(g):
        b, k = g // 6, g % 6
        row_copy((b, k), [(b, j, k) for j in range(_J)])

    def diag(t):
        b, j = t // _J, t % _J
        row_copy((b, 6 + j), [(b, j, 6)])

    @pl.when(wid < 16)
    def _():
        for i in range(2):
            bgroup(wid * 2 + i)

    @pl.when(wid >= 16)
    def _():
        bgroup(32 + (wid - 16))
        for i in range(4):
            diag((wid - 16) * 4 + i)


def kernel(x):
    b, s, n, m, d = x.shape
    mesh = plsc.VectorSubcoreMesh(core_axis_name="c", subcore_axis_name="s")
    fn = pl.kernel(
        _sc_body,
        out_type=jax.ShapeDtypeStruct((b, _J, _K, n, m, d), x.dtype),
        mesh=mesh,
        compiler_params=pltpu.CompilerParams(use_tc_tiling_on_sc=True),
        scratch_types=[
            pltpu.VMEM((n, m, d), x.dtype),
            pltpu.SemaphoreType.DMA,
            pltpu.SemaphoreType.DMA,
        ],
    )
    return fn(x)
